# baseline (speedup 1.0000x reference)
import functools

import jax
import jax.numpy as jnp
from jax import lax
from jax.experimental import pallas as pl
from jax.experimental.pallas import tpu as pltpu
from jax.experimental.pallas import tpu_sc as plsc

N_ENTITIES = 1000000
N_RELATIONS = 1000
EMBED_DIM = 64
BATCH = 16384

NUM_WORKERS = 32
B_PER_W = BATCH // NUM_WORKERS
CH = 16
N_CHUNKS = B_PER_W // CH
N_PAIRS = N_CHUNKS // 2

TBLK = 8192


def _transpose_body(in_ref, out_ref):
    x = in_ref[...]
    xt = jax.lax.dot_general(x, jnp.eye(EMBED_DIM, dtype=x.dtype),
                             (((0,), (0,)), ((), ())),
                             precision=jax.lax.Precision.HIGHEST,
                             preferred_element_type=jnp.float32)
    out_ref[...] = jnp.concatenate(
        [xt, jnp.zeros((TBLK, 128 - EMBED_DIM), jnp.float32)], axis=1)


def _to_row_major(table_t, n_rows):
    n_pad = (n_rows + TBLK - 1) // TBLK * TBLK
    grid = n_pad // TBLK
    out = pl.pallas_call(
        _transpose_body,
        grid=(grid,),
        in_specs=[pl.BlockSpec((EMBED_DIM, TBLK), lambda i: (0, i))],
        out_specs=pl.BlockSpec((TBLK, 128), lambda i: (i, 0)),
        out_shape=jax.ShapeDtypeStruct((n_pad, 128), jnp.float32),
    )(table_t)
    return out


def _sc_body(heads_hbm, rels_hbm, tails_hbm, etab_hbm, rtab_hbm, out_hbm,
             h_idx, r_idx, t_idx, blocks, bidx, out_v, sem0, sem1):
    wid = lax.axis_index("s") * 2 + lax.axis_index("c")
    base = wid * B_PER_W

    pltpu.sync_copy(heads_hbm.at[pl.ds(base, B_PER_W)], h_idx)
    pltpu.sync_copy(rels_hbm.at[pl.ds(base, B_PER_W)], r_idx)
    pltpu.sync_copy(tails_hbm.at[pl.ds(base, B_PER_W)], t_idx)

    lane = lax.iota(jnp.int32, 16)
    sems = (sem0, sem1)

    def fire(c, buf, sem):
        bidx[3 * buf + 0] = h_idx[pl.ds(c * CH, 16)]
        bidx[3 * buf + 1] = t_idx[pl.ds(c * CH, 16)]
        bidx[3 * buf + 2] = r_idx[pl.ds(c * CH, 16)]
        pltpu.async_copy(etab_hbm.at[bidx.at[3 * buf + 0]],
                         blocks.at[buf, pl.ds(0, CH)], sem)
        pltpu.async_copy(etab_hbm.at[bidx.at[3 * buf + 1]],
                         blocks.at[buf, pl.ds(CH, CH)], sem)
        pltpu.async_copy(rtab_hbm.at[bidx.at[3 * buf + 2]],
                         blocks.at[buf, pl.ds(2 * CH, CH)], sem)

    def wait(buf, sem):
        pltpu.make_async_copy(etab_hbm.at[bidx.at[3 * buf + 0]],
                              blocks.at[buf, pl.ds(0, CH)], sem).wait()
        pltpu.make_async_copy(etab_hbm.at[bidx.at[3 * buf + 1]],
                              blocks.at[buf, pl.ds(CH, CH)], sem).wait()
        pltpu.make_async_copy(rtab_hbm.at[bidx.at[3 * buf + 2]],
                              blocks.at[buf, pl.ds(2 * CH, CH)], sem).wait()

    def compute(c, buf):
        bref = blocks.at[buf]
        ent_h = lane
        ent_t = lane + CH
        ent_r = lane + 2 * CH
        acc = jnp.zeros((16,), jnp.float32)
        for d in range(EMBED_DIM):
            dsplat = jnp.full((16,), d, jnp.int32)
            vh = plsc.load_gather(bref, [ent_h, dsplat])
            vt = plsc.load_gather(bref, [ent_t, dsplat])
            vr = plsc.load_gather(bref, [ent_r, dsplat])
            s = (vh + vr) - vt
            acc = acc + s * s
        out_v[pl.ds(c * CH, 16)] = -acc

    fire(0, 0, sems[0])

    def pair_body(j, _):
        c0 = 2 * j
        fire(c0 + 1, 1, sems[1])
        wait(0, sems[0])
        compute(c0, 0)

        @pl.when(j < N_PAIRS - 1)
        def _():
            fire(c0 + 2, 0, sems[0])

        wait(1, sems[1])
        compute(c0 + 1, 1)
        return 0

    lax.fori_loop(0, N_PAIRS, pair_body, 0)

    pltpu.sync_copy(out_v, out_hbm.at[pl.ds(base, B_PER_W)])


@jax.jit
def _score(heads, relations, tails, etab_t, rtab_t):
    etab_rm = _to_row_major(etab_t, N_ENTITIES)
    rtab_rm = _to_row_major(rtab_t, N_RELATIONS)
    mesh = plsc.VectorSubcoreMesh(core_axis_name="c", subcore_axis_name="s")
    f = functools.partial(
        pl.kernel,
        mesh=mesh,
        compiler_params=pltpu.CompilerParams(
            needs_layout_passes=False, use_tc_tiling_on_sc=True),
        out_type=jax.ShapeDtypeStruct((BATCH,), jnp.float32),
        scratch_types=[
            pltpu.VMEM((B_PER_W,), jnp.int32),
            pltpu.VMEM((B_PER_W,), jnp.int32),
            pltpu.VMEM((B_PER_W,), jnp.int32),
            pltpu.VMEM((2, 3 * CH, 128), jnp.float32),
            pltpu.VMEM((6, CH), jnp.int32),
            pltpu.VMEM((B_PER_W,), jnp.float32),
            pltpu.SemaphoreType.DMA,
            pltpu.SemaphoreType.DMA,
        ],
    )(_sc_body)
    return f(heads, relations, tails, etab_rm, rtab_rm)


def kernel(heads, relations, tails, entity_table, relation_table):
    return _score(heads.astype(jnp.int32), relations.astype(jnp.int32),
                  tails.astype(jnp.int32), entity_table.T, relation_table.T)


# speedup vs baseline: 1.5178x; 1.5178x over previous
import functools

import jax
import jax.numpy as jnp
from jax import lax
from jax.experimental import pallas as pl
from jax.experimental.pallas import tpu as pltpu
from jax.experimental.pallas import tpu_sc as plsc

N_ENTITIES = 1000000
N_RELATIONS = 1000
EMBED_DIM = 64
BATCH = 16384

NUM_WORKERS = 32
B_PER_W = BATCH // NUM_WORKERS
CH = 16
N_CHUNKS = B_PER_W // CH
N_PAIRS = N_CHUNKS // 2

TBLK = 8192


def _transpose_body(in_ref, out_ref):
    x = in_ref[...]
    xt = x.T
    out_ref[...] = jnp.concatenate(
        [xt, jnp.zeros((TBLK, 128 - EMBED_DIM), jnp.float32)], axis=1)


def _to_row_major(table_t, n_rows):
    n_pad = (n_rows + TBLK - 1) // TBLK * TBLK
    grid = n_pad // TBLK
    out = pl.pallas_call(
        _transpose_body,
        grid=(grid,),
        in_specs=[pl.BlockSpec((EMBED_DIM, TBLK), lambda i: (0, i))],
        out_specs=pl.BlockSpec((TBLK, 128), lambda i: (i, 0)),
        out_shape=jax.ShapeDtypeStruct((n_pad, 128), jnp.float32),
    )(table_t)
    return out


def _sc_body(heads_hbm, rels_hbm, tails_hbm, etab_hbm, rtab_hbm, out_hbm,
             h_idx, r_idx, t_idx, blocks, bidx, out_v, sem0, sem1):
    wid = lax.axis_index("s") * 2 + lax.axis_index("c")
    base = wid * B_PER_W

    pltpu.sync_copy(heads_hbm.at[pl.ds(base, B_PER_W)], h_idx)
    pltpu.sync_copy(rels_hbm.at[pl.ds(base, B_PER_W)], r_idx)
    pltpu.sync_copy(tails_hbm.at[pl.ds(base, B_PER_W)], t_idx)

    lane = lax.iota(jnp.int32, 16)
    sems = (sem0, sem1)

    def fire(c, buf, sem):
        bidx[3 * buf + 0] = h_idx[pl.ds(c * CH, 16)]
        bidx[3 * buf + 1] = t_idx[pl.ds(c * CH, 16)]
        bidx[3 * buf + 2] = r_idx[pl.ds(c * CH, 16)]
        pltpu.async_copy(etab_hbm.at[bidx.at[3 * buf + 0]],
                         blocks.at[buf, pl.ds(0, CH)], sem)
        pltpu.async_copy(etab_hbm.at[bidx.at[3 * buf + 1]],
                         blocks.at[buf, pl.ds(CH, CH)], sem)
        pltpu.async_copy(rtab_hbm.at[bidx.at[3 * buf + 2]],
                         blocks.at[buf, pl.ds(2 * CH, CH)], sem)

    def wait(buf, sem):
        pltpu.make_async_copy(etab_hbm.at[bidx.at[3 * buf + 0]],
                              blocks.at[buf, pl.ds(0, CH)], sem).wait()
        pltpu.make_async_copy(etab_hbm.at[bidx.at[3 * buf + 1]],
                              blocks.at[buf, pl.ds(CH, CH)], sem).wait()
        pltpu.make_async_copy(rtab_hbm.at[bidx.at[3 * buf + 2]],
                              blocks.at[buf, pl.ds(2 * CH, CH)], sem).wait()

    def compute(c, buf):
        bref = blocks.at[buf]
        ent_h = lane
        ent_t = lane + CH
        ent_r = lane + 2 * CH
        acc = jnp.zeros((16,), jnp.float32)
        for d in range(EMBED_DIM):
            dsplat = jnp.full((16,), d, jnp.int32)
            vh = plsc.load_gather(bref, [ent_h, dsplat])
            vt = plsc.load_gather(bref, [ent_t, dsplat])
            vr = plsc.load_gather(bref, [ent_r, dsplat])
            s = (vh + vr) - vt
            acc = acc + s * s
        out_v[pl.ds(c * CH, 16)] = -acc

    fire(0, 0, sems[0])

    def pair_body(j, _):
        c0 = 2 * j
        fire(c0 + 1, 1, sems[1])
        wait(0, sems[0])
        compute(c0, 0)

        @pl.when(j < N_PAIRS - 1)
        def _():
            fire(c0 + 2, 0, sems[0])

        wait(1, sems[1])
        compute(c0 + 1, 1)
        return 0

    lax.fori_loop(0, N_PAIRS, pair_body, 0)

    pltpu.sync_copy(out_v, out_hbm.at[pl.ds(base, B_PER_W)])


@jax.jit
def _score(heads, relations, tails, etab_t, rtab_t):
    etab_rm = _to_row_major(etab_t, N_ENTITIES)
    rtab_rm = _to_row_major(rtab_t, N_RELATIONS)
    mesh = plsc.VectorSubcoreMesh(core_axis_name="c", subcore_axis_name="s")
    f = functools.partial(
        pl.kernel,
        mesh=mesh,
        compiler_params=pltpu.CompilerParams(
            needs_layout_passes=False, use_tc_tiling_on_sc=True),
        out_type=jax.ShapeDtypeStruct((BATCH,), jnp.float32),
        scratch_types=[
            pltpu.VMEM((B_PER_W,), jnp.int32),
            pltpu.VMEM((B_PER_W,), jnp.int32),
            pltpu.VMEM((B_PER_W,), jnp.int32),
            pltpu.VMEM((2, 3 * CH, 128), jnp.float32),
            pltpu.VMEM((6, CH), jnp.int32),
            pltpu.VMEM((B_PER_W,), jnp.float32),
            pltpu.SemaphoreType.DMA,
            pltpu.SemaphoreType.DMA,
        ],
    )(_sc_body)
    return f(heads, relations, tails, etab_rm, rtab_rm)


def kernel(heads, relations, tails, entity_table, relation_table):
    return _score(heads.astype(jnp.int32), relations.astype(jnp.int32),
                  tails.astype(jnp.int32), entity_table.T, relation_table.T)
